# baseline (speedup 1.0000x reference)
import functools

import jax
import jax.numpy as jnp
from jax import lax
from jax.experimental import pallas as pl
from jax.experimental.pallas import tpu as pltpu
from jax.experimental.pallas import tpu_sc as plsc

N = 10000
E = 320000
D_IN = 128
D_HID = 16
D_OUT = 128

NC = 2
NS = 16
NW = NC * NS
C = 128
NCHUNK = 80
EPT = NCHUNK * C
E_PAD = EPT * NW
N_P = 10240
RPT = N_P // NS
G = 8
NF = N_P // G
W128 = G * D_HID

_f32 = jnp.float32
_i32 = jnp.int32

_mesh = plsc.VectorSubcoreMesh(core_axis_name="c", subcore_axis_name="s",
                               num_cores=NC, num_subcores=NS)
_sc_params = pltpu.CompilerParams(use_tc_tiling_on_sc=False)


NBUF = 4


@functools.partial(
    pl.kernel,
    out_type=jax.ShapeDtypeStruct((NC * N_P, D_HID), _f32),
    mesh=_mesh,
    scratch_types=[
        pltpu.VMEM((NCHUNK, C), _i32),
        pltpu.VMEM((NCHUNK, C), _i32),
        [pltpu.VMEM((C, D_HID), _f32) for _ in range(NBUF)],
        [pltpu.SemaphoreType.DMA for _ in range(NBUF)],
        pltpu.VMEM_SHARED((N_P, D_HID), _f32),
    ],
    compiler_params=_sc_params,
)
def _seg_sum(table_hbm, src_hbm, dst_hbm, out_hbm,
             src_v, dst_v, rows, sems, acc_sh):
    cid = lax.axis_index("c")
    sid = lax.axis_index("s")
    tid = cid * NS + sid

    @pl.loop(0, C)
    def _(i):
        rows[0][i, :] = jnp.zeros((D_HID,), _f32)

    @pl.loop(0, RPT // C)
    def _(k):
        pltpu.sync_copy(rows[0], acc_sh.at[pl.ds(sid * RPT + k * C, C)])

    pltpu.sync_copy(src_hbm.at[pl.ds(tid * NCHUNK, NCHUNK)], src_v)
    pltpu.sync_copy(dst_hbm.at[pl.ds(tid * NCHUNK, NCHUNK)], dst_v)
    plsc.subcore_barrier()

    for b in range(NBUF):
        pltpu.async_copy(table_hbm.at[src_v.at[b]], rows[b], sems[b])

    @pl.loop(0, NCHUNK, step=NBUF)
    def _(ci):
        for b in range(NBUF):
            pltpu.make_async_copy(
                table_hbm.at[src_v.at[ci + b]], rows[b], sems[b]).wait()
            pltpu.sync_copy(rows[b], acc_sh.at[dst_v.at[ci + b]], add=True)

            @pl.when(ci + b + NBUF < NCHUNK)
            def _():
                pltpu.async_copy(
                    table_hbm.at[src_v.at[ci + b + NBUF]], rows[b], sems[b])

    plsc.subcore_barrier()
    pltpu.sync_copy(acc_sh.at[pl.ds(sid * RPT, RPT)],
                    out_hbm.at[pl.ds(cid * N_P + sid * RPT, RPT)])


@functools.partial(
    pl.kernel,
    out_type=jax.ShapeDtypeStruct((NC * N_P, D_HID), _f32),
    mesh=_mesh,
    scratch_types=[
        pltpu.VMEM((NCHUNK, C), _i32),
        pltpu.VMEM((C, D_HID), _f32),
        pltpu.VMEM_SHARED((N_P, D_HID), _f32),
    ],
    compiler_params=_sc_params,
)
def _count(dst_hbm, out_hbm, dst_v, rows_v, acc_sh):
    cid = lax.axis_index("c")
    sid = lax.axis_index("s")
    tid = cid * NS + sid

    @pl.loop(0, C)
    def _(i):
        rows_v[i, :] = jnp.zeros((D_HID,), _f32)

    @pl.loop(0, RPT // C)
    def _(k):
        pltpu.sync_copy(rows_v, acc_sh.at[pl.ds(sid * RPT + k * C, C)])

    @pl.loop(0, C)
    def _(i):
        rows_v[i, :] = jnp.ones((D_HID,), _f32)

    pltpu.sync_copy(dst_hbm.at[pl.ds(tid * NCHUNK, NCHUNK)], dst_v)
    plsc.subcore_barrier()

    @pl.loop(0, NCHUNK)
    def _(ci):
        pltpu.sync_copy(rows_v, acc_sh.at[dst_v.at[ci]], add=True)

    plsc.subcore_barrier()
    pltpu.sync_copy(acc_sh.at[pl.ds(sid * RPT, RPT)],
                    out_hbm.at[pl.ds(cid * N_P + sid * RPT, RPT)])


EPB = 16384
NBLK = E_PAD // EPB


def _prep_body(e_ref, src_ref, dst_ref):
    i = pl.program_id(0)
    s = e_ref[0].reshape(C, C)
    d = e_ref[1].reshape(C, C)
    row = lax.broadcasted_iota(_i32, (C, C), 0)
    lane = lax.broadcasted_iota(_i32, (C, C), 1)
    eid = i * EPB + row * C + lane
    valid = eid < E
    src_ref[...] = jnp.where(valid, s, 0)
    dst_ref[...] = jnp.where(valid, d, N)


def _prep(edge_index):
    return pl.pallas_call(
        _prep_body,
        grid=(NBLK,),
        in_specs=[pl.BlockSpec((2, EPB), lambda i: (0, i))],
        out_specs=[pl.BlockSpec((C, C), lambda i: (i, 0)),
                   pl.BlockSpec((C, C), lambda i: (i, 0))],
        out_shape=[jax.ShapeDtypeStruct((NW * NCHUNK, C), _i32),
                   jax.ShapeDtypeStruct((NW * NCHUNK, C), _i32)],
    )(edge_index)


def _full(shape):
    return pl.BlockSpec(shape, lambda i: tuple(0 for _ in shape))


def _p2():
    return [pl.BlockSpec((NF, W128), lambda i: (0, 0)),
            pl.BlockSpec((NF, W128), lambda i: (1, 0))]


def _mm1_body(xg_ref, wbd_ref, o_ref):
    o_ref[...] = jnp.dot(xg_ref[...], wbd_ref[...], preferred_element_type=_f32)


def _matmul1(x_g, W_bd):
    return pl.pallas_call(
        _mm1_body,
        grid=(1,),
        in_specs=[_full((NF, G * D_IN)), _full((G * D_IN, W128))],
        out_specs=_full((NF, W128)),
        out_shape=jax.ShapeDtypeStruct((NF, W128), _f32),
    )(x_g, W_bd)


def _scale1_body(cnt0_ref, cnt1_ref, h1_ref, hs_ref, dinv_ref):
    dinv = lax.rsqrt(cnt0_ref[...] + cnt1_ref[...] + 1.0)
    dinv_ref[...] = dinv
    hs_ref[...] = dinv * h1_ref[...]


def _scale1(cnt_f, h1_f):
    return pl.pallas_call(
        _scale1_body,
        grid=(1,),
        in_specs=_p2() + [_full((NF, W128))],
        out_specs=[_full((NF, W128)), _full((NF, W128))],
        out_shape=[jax.ShapeDtypeStruct((NF, W128), _f32),
                   jax.ShapeDtypeStruct((NF, W128), _f32)],
    )(cnt_f, cnt_f, h1_f)


def _scale2_body(s10_ref, s11_ref, h1s_ref, dinv_ref, b1_ref, o_ref):
    out1 = (dinv_ref[...] * (s10_ref[...] + s11_ref[...] + h1s_ref[...])
            + b1_ref[...])
    o_ref[...] = dinv_ref[...] * jnp.maximum(out1, 0.0)


def _scale2(s1_f, h1s_f, dinv_f, b1_t):
    return pl.pallas_call(
        _scale2_body,
        grid=(1,),
        in_specs=_p2() + [_full((NF, W128)), _full((NF, W128)),
                          _full((1, W128))],
        out_specs=_full((NF, W128)),
        out_shape=jax.ShapeDtypeStruct((NF, W128), _f32),
    )(s1_f, s1_f, h1s_f, dinv_f, b1_t)


def _final_body(s20_ref, s21_ref, hs_ref, dinv_ref, w2bd_ref, b2_ref, o_ref):
    u = dinv_ref[...] * (s20_ref[...] + s21_ref[...] + hs_ref[...])
    o_ref[...] = (jnp.dot(u, w2bd_ref[...], preferred_element_type=_f32)
                  + b2_ref[...])


def _final(s2_f, hs_f, dinv_f, W2_bd, b2_t):
    return pl.pallas_call(
        _final_body,
        grid=(1,),
        in_specs=_p2() + [_full((NF, W128)), _full((NF, W128)),
                          _full((W128, G * D_OUT)), _full((1, G * D_OUT))],
        out_specs=_full((NF, G * D_OUT)),
        out_shape=jax.ShapeDtypeStruct((NF, G * D_OUT), _f32),
    )(s2_f, s2_f, hs_f, dinv_f, W2_bd, b2_t)


def _block_diag(W, g):
    a, b = W.shape
    eye = jnp.eye(g, dtype=W.dtype)
    return (eye[:, None, :, None] * W[None, :, None, :]).reshape(g * a, g * b)


def kernel(x, edge_index, W1, b1, W2, b2):
    x = x.astype(_f32)
    src_p, dst_p = _prep(edge_index.astype(_i32))
    x_g = jnp.pad(x.reshape(N // G, G * D_IN), ((0, NF - N // G), (0, 0)))
    W_bd = _block_diag(W1, G)
    W2_bd = _block_diag(W2, G)
    b1_t = jnp.tile(b1, G).reshape(1, W128)
    b2_t = jnp.tile(b2, G).reshape(1, G * D_OUT)

    cnt = _count(dst_p)
    cnt_f = cnt.reshape(2 * NF, W128)
    h1_f = _matmul1(x_g, W_bd)
    h1s_f, dinv_f = _scale1(cnt_f, h1_f)
    s1 = _seg_sum(h1s_f.reshape(N_P, D_HID), src_p, dst_p)
    hs_f = _scale2(s1.reshape(2 * NF, W128), h1s_f, dinv_f, b1_t)
    s2 = _seg_sum(hs_f.reshape(N_P, D_HID), src_p, dst_p)
    out_g = _final(s2.reshape(2 * NF, W128), hs_f, dinv_f, W2_bd, b2_t)
    return out_g.reshape(N_P, D_OUT)[:N]


# speedup vs baseline: 1.0080x; 1.0080x over previous
import functools

import jax
import jax.numpy as jnp
from jax import lax
from jax.experimental import pallas as pl
from jax.experimental.pallas import tpu as pltpu
from jax.experimental.pallas import tpu_sc as plsc

N = 10000
E = 320000
D_IN = 128
D_HID = 16
D_OUT = 128

NC = 2
NS = 16
NW = NC * NS
C = 128
NCHUNK = 80
EPT = NCHUNK * C
E_PAD = EPT * NW
N_P = 10240
RPT = N_P // NS
G = 8
NF = N_P // G
W128 = G * D_HID

_f32 = jnp.float32
_i32 = jnp.int32

_mesh = plsc.VectorSubcoreMesh(core_axis_name="c", subcore_axis_name="s",
                               num_cores=NC, num_subcores=NS)
_sc_params = pltpu.CompilerParams(use_tc_tiling_on_sc=False)


NBUF = 2


@functools.partial(
    pl.kernel,
    out_type=jax.ShapeDtypeStruct((NC * N_P, D_HID), _f32),
    mesh=_mesh,
    scratch_types=[
        pltpu.VMEM((NCHUNK, C), _i32),
        pltpu.VMEM((NCHUNK, C), _i32),
        [pltpu.VMEM((C, D_HID), _f32) for _ in range(NBUF)],
        [pltpu.SemaphoreType.DMA for _ in range(NBUF)],
        pltpu.VMEM_SHARED((N_P, D_HID), _f32),
    ],
    compiler_params=_sc_params,
)
def _seg_sum(table_hbm, src_hbm, dst_hbm, out_hbm,
             src_v, dst_v, rows, sems, acc_sh):
    cid = lax.axis_index("c")
    sid = lax.axis_index("s")
    tid = cid * NS + sid

    @pl.loop(0, C)
    def _(i):
        rows[0][i, :] = jnp.zeros((D_HID,), _f32)

    @pl.loop(0, RPT // C)
    def _(k):
        pltpu.sync_copy(rows[0], acc_sh.at[pl.ds(sid * RPT + k * C, C)])

    pltpu.sync_copy(src_hbm.at[pl.ds(tid * NCHUNK, NCHUNK)], src_v)
    pltpu.sync_copy(dst_hbm.at[pl.ds(tid * NCHUNK, NCHUNK)], dst_v)
    plsc.subcore_barrier()

    for b in range(NBUF):
        pltpu.async_copy(table_hbm.at[src_v.at[b]], rows[b], sems[b])

    @pl.loop(0, NCHUNK, step=NBUF)
    def _(ci):
        for b in range(NBUF):
            pltpu.make_async_copy(
                table_hbm.at[src_v.at[ci + b]], rows[b], sems[b]).wait()
            pltpu.sync_copy(rows[b], acc_sh.at[dst_v.at[ci + b]], add=True)

            @pl.when(ci + b + NBUF < NCHUNK)
            def _():
                pltpu.async_copy(
                    table_hbm.at[src_v.at[ci + b + NBUF]], rows[b], sems[b])

    plsc.subcore_barrier()
    pltpu.sync_copy(acc_sh.at[pl.ds(sid * RPT, RPT)],
                    out_hbm.at[pl.ds(cid * N_P + sid * RPT, RPT)])


@functools.partial(
    pl.kernel,
    out_type=jax.ShapeDtypeStruct((NC * N_P, D_HID), _f32),
    mesh=_mesh,
    scratch_types=[
        pltpu.VMEM((NCHUNK, C), _i32),
        pltpu.VMEM((C, D_HID), _f32),
        pltpu.VMEM_SHARED((N_P, D_HID), _f32),
    ],
    compiler_params=_sc_params,
)
def _count(dst_hbm, out_hbm, dst_v, rows_v, acc_sh):
    cid = lax.axis_index("c")
    sid = lax.axis_index("s")
    tid = cid * NS + sid

    @pl.loop(0, C)
    def _(i):
        rows_v[i, :] = jnp.zeros((D_HID,), _f32)

    @pl.loop(0, RPT // C)
    def _(k):
        pltpu.sync_copy(rows_v, acc_sh.at[pl.ds(sid * RPT + k * C, C)])

    @pl.loop(0, C)
    def _(i):
        rows_v[i, :] = jnp.ones((D_HID,), _f32)

    pltpu.sync_copy(dst_hbm.at[pl.ds(tid * NCHUNK, NCHUNK)], dst_v)
    plsc.subcore_barrier()

    @pl.loop(0, NCHUNK)
    def _(ci):
        pltpu.sync_copy(rows_v, acc_sh.at[dst_v.at[ci]], add=True)

    plsc.subcore_barrier()
    pltpu.sync_copy(acc_sh.at[pl.ds(sid * RPT, RPT)],
                    out_hbm.at[pl.ds(cid * N_P + sid * RPT, RPT)])


EPB = 81920
NBLK = E_PAD // EPB


def _prep_body(e_ref, src_ref, dst_ref):
    i = pl.program_id(0)
    s = e_ref[0].reshape(EPB // C, C)
    d = e_ref[1].reshape(EPB // C, C)
    row = lax.broadcasted_iota(_i32, (EPB // C, C), 0)
    lane = lax.broadcasted_iota(_i32, (EPB // C, C), 1)
    eid = i * EPB + row * C + lane
    valid = eid < E
    src_ref[...] = jnp.where(valid, s, 0)
    dst_ref[...] = jnp.where(valid, d, N)


def _prep(edge_index):
    return pl.pallas_call(
        _prep_body,
        grid=(NBLK,),
        in_specs=[pl.BlockSpec((2, EPB), lambda i: (0, i))],
        out_specs=[pl.BlockSpec((EPB // C, C), lambda i: (i, 0)),
                   pl.BlockSpec((EPB // C, C), lambda i: (i, 0))],
        out_shape=[jax.ShapeDtypeStruct((NW * NCHUNK, C), _i32),
                   jax.ShapeDtypeStruct((NW * NCHUNK, C), _i32)],
    )(edge_index)


def _full(shape):
    return pl.BlockSpec(shape, lambda i: tuple(0 for _ in shape))


def _p2():
    return [pl.BlockSpec((NF, W128), lambda i: (0, 0)),
            pl.BlockSpec((NF, W128), lambda i: (1, 0))]


def _mm1_body(xg_ref, wbd_ref, o_ref):
    o_ref[...] = jnp.dot(xg_ref[...], wbd_ref[...], preferred_element_type=_f32)


def _matmul1(x_g, W_bd):
    return pl.pallas_call(
        _mm1_body,
        grid=(1,),
        in_specs=[_full((NF, G * D_IN)), _full((G * D_IN, W128))],
        out_specs=_full((NF, W128)),
        out_shape=jax.ShapeDtypeStruct((NF, W128), _f32),
    )(x_g, W_bd)


def _scale1_body(cnt0_ref, cnt1_ref, h1_ref, hs_ref, dinv_ref):
    dinv = lax.rsqrt(cnt0_ref[...] + cnt1_ref[...] + 1.0)
    dinv_ref[...] = dinv
    hs_ref[...] = dinv * h1_ref[...]


def _scale1(cnt_f, h1_f):
    return pl.pallas_call(
        _scale1_body,
        grid=(1,),
        in_specs=_p2() + [_full((NF, W128))],
        out_specs=[_full((NF, W128)), _full((NF, W128))],
        out_shape=[jax.ShapeDtypeStruct((NF, W128), _f32),
                   jax.ShapeDtypeStruct((NF, W128), _f32)],
    )(cnt_f, cnt_f, h1_f)


def _scale2_body(s10_ref, s11_ref, h1s_ref, dinv_ref, b1_ref, o_ref):
    out1 = (dinv_ref[...] * (s10_ref[...] + s11_ref[...] + h1s_ref[...])
            + b1_ref[...])
    o_ref[...] = dinv_ref[...] * jnp.maximum(out1, 0.0)


def _scale2(s1_f, h1s_f, dinv_f, b1_t):
    return pl.pallas_call(
        _scale2_body,
        grid=(1,),
        in_specs=_p2() + [_full((NF, W128)), _full((NF, W128)),
                          _full((1, W128))],
        out_specs=_full((NF, W128)),
        out_shape=jax.ShapeDtypeStruct((NF, W128), _f32),
    )(s1_f, s1_f, h1s_f, dinv_f, b1_t)


def _final_body(s20_ref, s21_ref, hs_ref, dinv_ref, w2bd_ref, b2_ref, o_ref):
    u = dinv_ref[...] * (s20_ref[...] + s21_ref[...] + hs_ref[...])
    o_ref[...] = (jnp.dot(u, w2bd_ref[...], preferred_element_type=_f32)
                  + b2_ref[...])


def _final(s2_f, hs_f, dinv_f, W2_bd, b2_t):
    return pl.pallas_call(
        _final_body,
        grid=(1,),
        in_specs=_p2() + [_full((NF, W128)), _full((NF, W128)),
                          _full((W128, G * D_OUT)), _full((1, G * D_OUT))],
        out_specs=_full((NF, G * D_OUT)),
        out_shape=jax.ShapeDtypeStruct((NF, G * D_OUT), _f32),
    )(s2_f, s2_f, hs_f, dinv_f, W2_bd, b2_t)


def _block_diag(W, g):
    a, b = W.shape
    eye = jnp.eye(g, dtype=W.dtype)
    return (eye[:, None, :, None] * W[None, :, None, :]).reshape(g * a, g * b)


def kernel(x, edge_index, W1, b1, W2, b2):
    x = x.astype(_f32)
    src_p, dst_p = _prep(edge_index.astype(_i32))
    x_g = jnp.pad(x.reshape(N // G, G * D_IN), ((0, NF - N // G), (0, 0)))
    W_bd = _block_diag(W1, G)
    W2_bd = _block_diag(W2, G)
    b1_t = jnp.tile(b1, G).reshape(1, W128)
    b2_t = jnp.tile(b2, G).reshape(1, G * D_OUT)

    cnt = _count(dst_p)
    cnt_f = cnt.reshape(2 * NF, W128)
    h1_f = _matmul1(x_g, W_bd)
    h1s_f, dinv_f = _scale1(cnt_f, h1_f)
    s1 = _seg_sum(h1s_f.reshape(N_P, D_HID), src_p, dst_p)
    hs_f = _scale2(s1.reshape(2 * NF, W128), h1s_f, dinv_f, b1_t)
    s2 = _seg_sum(hs_f.reshape(N_P, D_HID), src_p, dst_p)
    out_g = _final(s2.reshape(2 * NF, W128), hs_f, dinv_f, W2_bd, b2_t)
    return out_g.reshape(N_P, D_OUT)[:N]


# speedup vs baseline: 1.3581x; 1.3472x over previous
import functools

import jax
import jax.numpy as jnp
from jax import lax
from jax.experimental import pallas as pl
from jax.experimental.pallas import tpu as pltpu
from jax.experimental.pallas import tpu_sc as plsc

N = 10000
E = 320000
D_IN = 128
D_HID = 16
D_OUT = 128

NC = 2
NS = 16
NW = NC * NS
C = 128
NCHUNK = 80
EPT = NCHUNK * C
E_PAD = EPT * NW
N_P = 10240
RPT = N_P // NS
G = 8
NF = N_P // G
W128 = G * D_HID

_f32 = jnp.float32
_i32 = jnp.int32

_mesh = plsc.VectorSubcoreMesh(core_axis_name="c", subcore_axis_name="s",
                               num_cores=NC, num_subcores=NS)
_sc_params = pltpu.CompilerParams(use_tc_tiling_on_sc=False)


NBUF = 2


@functools.partial(
    pl.kernel,
    out_type=jax.ShapeDtypeStruct((NC * N_P, D_HID), _f32),
    mesh=_mesh,
    scratch_types=[
        pltpu.VMEM((NCHUNK, C), _i32),
        pltpu.VMEM((NCHUNK, C), _i32),
        [pltpu.VMEM((C, D_HID), _f32) for _ in range(NBUF)],
        [pltpu.SemaphoreType.DMA for _ in range(NBUF)],
        pltpu.VMEM_SHARED((N_P, D_HID), _f32),
    ],
    compiler_params=_sc_params,
)
def _seg_sum(table_hbm, src_hbm, dst_hbm, out_hbm,
             src_v, dst_v, rows, sems, acc_sh):
    cid = lax.axis_index("c")
    sid = lax.axis_index("s")
    tid = cid * NS + sid

    @pl.loop(0, C)
    def _(i):
        rows[0][i, :] = jnp.zeros((D_HID,), _f32)

    @pl.loop(0, RPT // C)
    def _(k):
        pltpu.sync_copy(rows[0], acc_sh.at[pl.ds(sid * RPT + k * C, C)])

    pltpu.sync_copy(src_hbm.at[pl.ds(tid * NCHUNK, NCHUNK)], src_v)
    pltpu.sync_copy(dst_hbm.at[pl.ds(tid * NCHUNK, NCHUNK)], dst_v)
    plsc.subcore_barrier()

    for b in range(NBUF):
        pltpu.async_copy(table_hbm.at[src_v.at[b]], rows[b], sems[b])

    @pl.loop(0, NCHUNK, step=NBUF)
    def _(ci):
        for b in range(NBUF):
            pltpu.make_async_copy(
                table_hbm.at[src_v.at[ci + b]], rows[b], sems[b]).wait()
            pltpu.sync_copy(rows[b], acc_sh.at[dst_v.at[ci + b]], add=True)

            @pl.when(ci + b + NBUF < NCHUNK)
            def _():
                pltpu.async_copy(
                    table_hbm.at[src_v.at[ci + b + NBUF]], rows[b], sems[b])

    plsc.subcore_barrier()
    pltpu.sync_copy(acc_sh.at[pl.ds(sid * RPT, RPT)],
                    out_hbm.at[pl.ds(cid * N_P + sid * RPT, RPT)])


@functools.partial(
    pl.kernel,
    out_type=jax.ShapeDtypeStruct((NC * N_P, D_HID), _f32),
    mesh=_mesh,
    scratch_types=[
        pltpu.VMEM((NCHUNK, C), _i32),
        pltpu.VMEM((C, D_HID), _f32),
        pltpu.VMEM_SHARED((N_P, D_HID), _f32),
    ],
    compiler_params=_sc_params,
)
def _count(dst_hbm, out_hbm, dst_v, rows_v, acc_sh):
    cid = lax.axis_index("c")
    sid = lax.axis_index("s")
    tid = cid * NS + sid

    @pl.loop(0, C)
    def _(i):
        rows_v[i, :] = jnp.zeros((D_HID,), _f32)

    @pl.loop(0, RPT // C)
    def _(k):
        pltpu.sync_copy(rows_v, acc_sh.at[pl.ds(sid * RPT + k * C, C)])

    @pl.loop(0, C)
    def _(i):
        rows_v[i, :] = jnp.ones((D_HID,), _f32)

    pltpu.sync_copy(dst_hbm.at[pl.ds(tid * NCHUNK, NCHUNK)], dst_v)
    plsc.subcore_barrier()

    @pl.loop(0, NCHUNK)
    def _(ci):
        pltpu.sync_copy(rows_v, acc_sh.at[dst_v.at[ci]], add=True)

    plsc.subcore_barrier()
    pltpu.sync_copy(acc_sh.at[pl.ds(sid * RPT, RPT)],
                    out_hbm.at[pl.ds(cid * N_P + sid * RPT, RPT)])


EPB = 81920
NBLK = E_PAD // EPB


def _prep_body(e_ref, src_ref, dst_ref):
    i = pl.program_id(0)
    s = e_ref[0].reshape(EPB // C, C)
    d = e_ref[1].reshape(EPB // C, C)
    row = lax.broadcasted_iota(_i32, (EPB // C, C), 0)
    lane = lax.broadcasted_iota(_i32, (EPB // C, C), 1)
    eid = i * EPB + row * C + lane
    valid = eid < E
    src_ref[...] = jnp.where(valid, s, eid & 8191)
    dst_ref[...] = jnp.where(valid, d, N + (eid & 127))


def _prep(edge_index):
    return pl.pallas_call(
        _prep_body,
        grid=(NBLK,),
        in_specs=[pl.BlockSpec((2, EPB), lambda i: (0, i))],
        out_specs=[pl.BlockSpec((EPB // C, C), lambda i: (i, 0)),
                   pl.BlockSpec((EPB // C, C), lambda i: (i, 0))],
        out_shape=[jax.ShapeDtypeStruct((NW * NCHUNK, C), _i32),
                   jax.ShapeDtypeStruct((NW * NCHUNK, C), _i32)],
    )(edge_index)


def _full(shape):
    return pl.BlockSpec(shape, lambda i: tuple(0 for _ in shape))


def _p2():
    return [pl.BlockSpec((NF, W128), lambda i: (0, 0)),
            pl.BlockSpec((NF, W128), lambda i: (1, 0))]


def _mm1_body(xg_ref, wbd_ref, o_ref):
    o_ref[...] = jnp.dot(xg_ref[...], wbd_ref[...], preferred_element_type=_f32)


def _matmul1(x_g, W_bd):
    return pl.pallas_call(
        _mm1_body,
        grid=(1,),
        in_specs=[_full((NF, G * D_IN)), _full((G * D_IN, W128))],
        out_specs=_full((NF, W128)),
        out_shape=jax.ShapeDtypeStruct((NF, W128), _f32),
    )(x_g, W_bd)


def _scale1_body(cnt0_ref, cnt1_ref, h1_ref, hs_ref, dinv_ref):
    dinv = lax.rsqrt(cnt0_ref[...] + cnt1_ref[...] + 1.0)
    dinv_ref[...] = dinv
    hs_ref[...] = dinv * h1_ref[...]


def _scale1(cnt_f, h1_f):
    return pl.pallas_call(
        _scale1_body,
        grid=(1,),
        in_specs=_p2() + [_full((NF, W128))],
        out_specs=[_full((NF, W128)), _full((NF, W128))],
        out_shape=[jax.ShapeDtypeStruct((NF, W128), _f32),
                   jax.ShapeDtypeStruct((NF, W128), _f32)],
    )(cnt_f, cnt_f, h1_f)


def _scale2_body(s10_ref, s11_ref, h1s_ref, dinv_ref, b1_ref, o_ref):
    out1 = (dinv_ref[...] * (s10_ref[...] + s11_ref[...] + h1s_ref[...])
            + b1_ref[...])
    o_ref[...] = dinv_ref[...] * jnp.maximum(out1, 0.0)


def _scale2(s1_f, h1s_f, dinv_f, b1_t):
    return pl.pallas_call(
        _scale2_body,
        grid=(1,),
        in_specs=_p2() + [_full((NF, W128)), _full((NF, W128)),
                          _full((1, W128))],
        out_specs=_full((NF, W128)),
        out_shape=jax.ShapeDtypeStruct((NF, W128), _f32),
    )(s1_f, s1_f, h1s_f, dinv_f, b1_t)


def _final_body(s20_ref, s21_ref, hs_ref, dinv_ref, w2bd_ref, b2_ref, o_ref):
    u = dinv_ref[...] * (s20_ref[...] + s21_ref[...] + hs_ref[...])
    o_ref[...] = (jnp.dot(u, w2bd_ref[...], preferred_element_type=_f32)
                  + b2_ref[...])


def _final(s2_f, hs_f, dinv_f, W2_bd, b2_t):
    return pl.pallas_call(
        _final_body,
        grid=(1,),
        in_specs=_p2() + [_full((NF, W128)), _full((NF, W128)),
                          _full((W128, G * D_OUT)), _full((1, G * D_OUT))],
        out_specs=_full((NF, G * D_OUT)),
        out_shape=jax.ShapeDtypeStruct((NF, G * D_OUT), _f32),
    )(s2_f, s2_f, hs_f, dinv_f, W2_bd, b2_t)


def _block_diag(W, g):
    a, b = W.shape
    eye = jnp.eye(g, dtype=W.dtype)
    return (eye[:, None, :, None] * W[None, :, None, :]).reshape(g * a, g * b)


def kernel(x, edge_index, W1, b1, W2, b2):
    x = x.astype(_f32)
    src_p, dst_p = _prep(edge_index.astype(_i32))
    x_g = jnp.pad(x.reshape(N // G, G * D_IN), ((0, NF - N // G), (0, 0)))
    W_bd = _block_diag(W1, G)
    W2_bd = _block_diag(W2, G)
    b1_t = jnp.tile(b1, G).reshape(1, W128)
    b2_t = jnp.tile(b2, G).reshape(1, G * D_OUT)

    cnt = _count(dst_p)
    cnt_f = cnt.reshape(2 * NF, W128)
    h1_f = _matmul1(x_g, W_bd)
    h1s_f, dinv_f = _scale1(cnt_f, h1_f)
    s1 = _seg_sum(h1s_f.reshape(N_P, D_HID), src_p, dst_p)
    hs_f = _scale2(s1.reshape(2 * NF, W128), h1s_f, dinv_f, b1_t)
    s2 = _seg_sum(hs_f.reshape(N_P, D_HID), src_p, dst_p)
    out_g = _final(s2.reshape(2 * NF, W128), hs_f, dinv_f, W2_bd, b2_t)
    return out_g.reshape(N_P, D_OUT)[:N]


# speedup vs baseline: 1.4562x; 1.0723x over previous
import functools

import jax
import jax.numpy as jnp
from jax import lax
from jax.experimental import pallas as pl
from jax.experimental.pallas import tpu as pltpu
from jax.experimental.pallas import tpu_sc as plsc

N = 10000
E = 320000
D_IN = 128
D_HID = 16
D_OUT = 128

NC = 2
NS = 16
NW = NC * NS
C = 128
NCHUNK = 80
EPT = NCHUNK * C
E_PAD = EPT * NW
N_P = 10240
RPT = N_P // NS
G = 8
NF = N_P // G
W128 = G * D_HID

_f32 = jnp.float32
_i32 = jnp.int32

_mesh = plsc.VectorSubcoreMesh(core_axis_name="c", subcore_axis_name="s",
                               num_cores=NC, num_subcores=NS)
_sc_params = pltpu.CompilerParams(use_tc_tiling_on_sc=False)


NBUF = 4


@functools.partial(
    pl.kernel,
    out_type=jax.ShapeDtypeStruct((NC * N_P, D_HID), _f32),
    mesh=_mesh,
    scratch_types=[
        pltpu.VMEM((NCHUNK, C), _i32),
        pltpu.VMEM((NCHUNK, C), _i32),
        [pltpu.VMEM((C, D_HID), _f32) for _ in range(NBUF)],
        [pltpu.SemaphoreType.DMA for _ in range(NBUF)],
        [pltpu.SemaphoreType.DMA for _ in range(NBUF)],
        pltpu.VMEM_SHARED((N_P, D_HID), _f32),
    ],
    compiler_params=_sc_params,
)
def _seg_sum(table_hbm, src_hbm, dst_hbm, out_hbm,
             src_v, dst_v, rows, gsem, ssem, acc_sh):
    cid = lax.axis_index("c")
    sid = lax.axis_index("s")
    tid = cid * NS + sid

    @pl.loop(0, C)
    def _(i):
        rows[0][i, :] = jnp.zeros((D_HID,), _f32)

    @pl.loop(0, RPT // C)
    def _(k):
        pltpu.sync_copy(rows[0], acc_sh.at[pl.ds(sid * RPT + k * C, C)])

    pltpu.sync_copy(src_hbm.at[pl.ds(tid * NCHUNK, NCHUNK)], src_v)
    pltpu.sync_copy(dst_hbm.at[pl.ds(tid * NCHUNK, NCHUNK)], dst_v)
    plsc.subcore_barrier()

    def gather(j, b):
        pltpu.async_copy(table_hbm.at[src_v.at[j]], rows[b], gsem[b])

    def wait_gather(j, b):
        pltpu.make_async_copy(table_hbm.at[src_v.at[j]], rows[b],
                              gsem[b]).wait()

    def scatter(j, b):
        pltpu.async_copy(rows[b], acc_sh.at[dst_v.at[j]], ssem[b], add=True)

    def wait_scatter(j, b):
        pltpu.make_async_copy(rows[b], acc_sh.at[dst_v.at[j]],
                              ssem[b]).wait()

    gather(0, 0)
    gather(1, 1)
    wait_gather(0, 0)
    scatter(0, 0)
    gather(2, 2)
    wait_gather(1, 1)
    scatter(1, 1)
    gather(3, 3)

    @pl.loop(2, NCHUNK - 2, step=NBUF)
    def _(j0):
        for b in range(NBUF):
            j = j0 + b
            buf = (2 + b) % NBUF
            wait_gather(j, buf)
            scatter(j, buf)
            bg = b % NBUF
            wait_scatter(j - 2, bg)
            gather(j + 2, bg)

    wait_gather(NCHUNK - 2, (NCHUNK - 2) % NBUF)
    scatter(NCHUNK - 2, (NCHUNK - 2) % NBUF)
    wait_gather(NCHUNK - 1, (NCHUNK - 1) % NBUF)
    scatter(NCHUNK - 1, (NCHUNK - 1) % NBUF)
    for j in range(NCHUNK - 4, NCHUNK):
        wait_scatter(j, j % NBUF)

    plsc.subcore_barrier()
    pltpu.sync_copy(acc_sh.at[pl.ds(sid * RPT, RPT)],
                    out_hbm.at[pl.ds(cid * N_P + sid * RPT, RPT)])


@functools.partial(
    pl.kernel,
    out_type=jax.ShapeDtypeStruct((NC * N_P, D_HID), _f32),
    mesh=_mesh,
    scratch_types=[
        pltpu.VMEM((NCHUNK, C), _i32),
        pltpu.VMEM((C, D_HID), _f32),
        pltpu.VMEM_SHARED((N_P, D_HID), _f32),
    ],
    compiler_params=_sc_params,
)
def _count(dst_hbm, out_hbm, dst_v, rows_v, acc_sh):
    cid = lax.axis_index("c")
    sid = lax.axis_index("s")
    tid = cid * NS + sid

    @pl.loop(0, C)
    def _(i):
        rows_v[i, :] = jnp.zeros((D_HID,), _f32)

    @pl.loop(0, RPT // C)
    def _(k):
        pltpu.sync_copy(rows_v, acc_sh.at[pl.ds(sid * RPT + k * C, C)])

    @pl.loop(0, C)
    def _(i):
        rows_v[i, :] = jnp.ones((D_HID,), _f32)

    pltpu.sync_copy(dst_hbm.at[pl.ds(tid * NCHUNK, NCHUNK)], dst_v)
    plsc.subcore_barrier()

    @pl.loop(0, NCHUNK)
    def _(ci):
        pltpu.sync_copy(rows_v, acc_sh.at[dst_v.at[ci]], add=True)

    plsc.subcore_barrier()
    pltpu.sync_copy(acc_sh.at[pl.ds(sid * RPT, RPT)],
                    out_hbm.at[pl.ds(cid * N_P + sid * RPT, RPT)])


EPB = 81920
NBLK = E_PAD // EPB


def _prep_body(e_ref, src_ref, dst_ref):
    i = pl.program_id(0)
    s = e_ref[0].reshape(EPB // C, C)
    d = e_ref[1].reshape(EPB // C, C)
    row = lax.broadcasted_iota(_i32, (EPB // C, C), 0)
    lane = lax.broadcasted_iota(_i32, (EPB // C, C), 1)
    eid = i * EPB + row * C + lane
    valid = eid < E
    src_ref[...] = jnp.where(valid, s, eid & 8191)
    dst_ref[...] = jnp.where(valid, d, N + (eid & 127))


def _prep(edge_index):
    return pl.pallas_call(
        _prep_body,
        grid=(NBLK,),
        in_specs=[pl.BlockSpec((2, EPB), lambda i: (0, i))],
        out_specs=[pl.BlockSpec((EPB // C, C), lambda i: (i, 0)),
                   pl.BlockSpec((EPB // C, C), lambda i: (i, 0))],
        out_shape=[jax.ShapeDtypeStruct((NW * NCHUNK, C), _i32),
                   jax.ShapeDtypeStruct((NW * NCHUNK, C), _i32)],
    )(edge_index)


def _full(shape):
    return pl.BlockSpec(shape, lambda i: tuple(0 for _ in shape))


def _p2():
    return [pl.BlockSpec((NF, W128), lambda i: (0, 0)),
            pl.BlockSpec((NF, W128), lambda i: (1, 0))]


def _mm1_body(xg_ref, wbd_ref, o_ref):
    o_ref[...] = jnp.dot(xg_ref[...], wbd_ref[...], preferred_element_type=_f32)


def _matmul1(x_g, W_bd):
    return pl.pallas_call(
        _mm1_body,
        grid=(1,),
        in_specs=[_full((NF, G * D_IN)), _full((G * D_IN, W128))],
        out_specs=_full((NF, W128)),
        out_shape=jax.ShapeDtypeStruct((NF, W128), _f32),
    )(x_g, W_bd)


def _scale1_body(cnt0_ref, cnt1_ref, h1_ref, hs_ref, dinv_ref):
    dinv = lax.rsqrt(cnt0_ref[...] + cnt1_ref[...] + 1.0)
    dinv_ref[...] = dinv
    hs_ref[...] = dinv * h1_ref[...]


def _scale1(cnt_f, h1_f):
    return pl.pallas_call(
        _scale1_body,
        grid=(1,),
        in_specs=_p2() + [_full((NF, W128))],
        out_specs=[_full((NF, W128)), _full((NF, W128))],
        out_shape=[jax.ShapeDtypeStruct((NF, W128), _f32),
                   jax.ShapeDtypeStruct((NF, W128), _f32)],
    )(cnt_f, cnt_f, h1_f)


def _scale2_body(s10_ref, s11_ref, h1s_ref, dinv_ref, b1_ref, o_ref):
    out1 = (dinv_ref[...] * (s10_ref[...] + s11_ref[...] + h1s_ref[...])
            + b1_ref[...])
    o_ref[...] = dinv_ref[...] * jnp.maximum(out1, 0.0)


def _scale2(s1_f, h1s_f, dinv_f, b1_t):
    return pl.pallas_call(
        _scale2_body,
        grid=(1,),
        in_specs=_p2() + [_full((NF, W128)), _full((NF, W128)),
                          _full((1, W128))],
        out_specs=_full((NF, W128)),
        out_shape=jax.ShapeDtypeStruct((NF, W128), _f32),
    )(s1_f, s1_f, h1s_f, dinv_f, b1_t)


def _final_body(s20_ref, s21_ref, hs_ref, dinv_ref, w2bd_ref, b2_ref, o_ref):
    u = dinv_ref[...] * (s20_ref[...] + s21_ref[...] + hs_ref[...])
    o_ref[...] = (jnp.dot(u, w2bd_ref[...], preferred_element_type=_f32)
                  + b2_ref[...])


def _final(s2_f, hs_f, dinv_f, W2_bd, b2_t):
    return pl.pallas_call(
        _final_body,
        grid=(1,),
        in_specs=_p2() + [_full((NF, W128)), _full((NF, W128)),
                          _full((W128, G * D_OUT)), _full((1, G * D_OUT))],
        out_specs=_full((NF, G * D_OUT)),
        out_shape=jax.ShapeDtypeStruct((NF, G * D_OUT), _f32),
    )(s2_f, s2_f, hs_f, dinv_f, W2_bd, b2_t)


def _block_diag(W, g):
    a, b = W.shape
    eye = jnp.eye(g, dtype=W.dtype)
    return (eye[:, None, :, None] * W[None, :, None, :]).reshape(g * a, g * b)


def kernel(x, edge_index, W1, b1, W2, b2):
    x = x.astype(_f32)
    src_p, dst_p = _prep(edge_index.astype(_i32))
    x_g = jnp.pad(x.reshape(N // G, G * D_IN), ((0, NF - N // G), (0, 0)))
    W_bd = _block_diag(W1, G)
    W2_bd = _block_diag(W2, G)
    b1_t = jnp.tile(b1, G).reshape(1, W128)
    b2_t = jnp.tile(b2, G).reshape(1, G * D_OUT)

    cnt = _count(dst_p)
    cnt_f = cnt.reshape(2 * NF, W128)
    h1_f = _matmul1(x_g, W_bd)
    h1s_f, dinv_f = _scale1(cnt_f, h1_f)
    s1 = _seg_sum(h1s_f.reshape(N_P, D_HID), src_p, dst_p)
    hs_f = _scale2(s1.reshape(2 * NF, W128), h1s_f, dinv_f, b1_t)
    s2 = _seg_sum(hs_f.reshape(N_P, D_HID), src_p, dst_p)
    out_g = _final(s2.reshape(2 * NF, W128), hs_f, dinv_f, W2_bd, b2_t)
    return out_g.reshape(N_P, D_OUT)[:N]


# speedup vs baseline: 1.8692x; 1.2836x over previous
import functools

import jax
import jax.numpy as jnp
from jax import lax
from jax.experimental import pallas as pl
from jax.experimental.pallas import tpu as pltpu
from jax.experimental.pallas import tpu_sc as plsc

N = 10000
E = 320000
D_IN = 128
D_HID = 16
D_OUT = 128

NC = 2
NS = 16
NW = NC * NS
C = 128
NCHUNK = 80
EPT = NCHUNK * C
E_PAD = EPT * NW
N_P = 10240
RPT = N_P // NS
G = 8
NF = N_P // G
W128 = G * D_HID

_f32 = jnp.float32
_i32 = jnp.int32

_mesh = plsc.VectorSubcoreMesh(core_axis_name="c", subcore_axis_name="s",
                               num_cores=NC, num_subcores=NS)
_sc_params = pltpu.CompilerParams(use_tc_tiling_on_sc=False)


NBUF = 8


@functools.partial(
    pl.kernel,
    out_type=jax.ShapeDtypeStruct((NC * N_P, D_HID), _f32),
    mesh=_mesh,
    scratch_types=[
        pltpu.VMEM((NCHUNK, C), _i32),
        pltpu.VMEM((NCHUNK, C), _i32),
        [pltpu.VMEM((C, D_HID), _f32) for _ in range(NBUF)],
        [pltpu.SemaphoreType.DMA for _ in range(NBUF)],
        [pltpu.SemaphoreType.DMA for _ in range(NBUF)],
        pltpu.VMEM_SHARED((N_P, D_HID), _f32),
    ],
    compiler_params=_sc_params,
)
def _seg_sum(table_hbm, src_hbm, dst_hbm, out_hbm,
             src_v, dst_v, rows, gsem, ssem, acc_sh):
    cid = lax.axis_index("c")
    sid = lax.axis_index("s")
    tid = cid * NS + sid

    @pl.loop(0, C)
    def _(i):
        rows[0][i, :] = jnp.zeros((D_HID,), _f32)

    @pl.loop(0, RPT // C)
    def _(k):
        pltpu.sync_copy(rows[0], acc_sh.at[pl.ds(sid * RPT + k * C, C)])

    pltpu.sync_copy(src_hbm.at[pl.ds(tid * NCHUNK, NCHUNK)], src_v)
    pltpu.sync_copy(dst_hbm.at[pl.ds(tid * NCHUNK, NCHUNK)], dst_v)
    plsc.subcore_barrier()

    def gather(j, b):
        pltpu.async_copy(table_hbm.at[src_v.at[j]], rows[b], gsem[b])

    def wait_gather(j, b):
        pltpu.make_async_copy(table_hbm.at[src_v.at[j]], rows[b],
                              gsem[b]).wait()

    def scatter(j, b):
        pltpu.async_copy(rows[b], acc_sh.at[dst_v.at[j]], ssem[b], add=True)

    def wait_scatter(j, b):
        pltpu.make_async_copy(rows[b], acc_sh.at[dst_v.at[j]],
                              ssem[b]).wait()

    D = NBUF // 2
    for j in range(D):
        gather(j, j)
    for j in range(D):
        wait_gather(j, j)
        scatter(j, j)
        gather(j + D, j + D)

    @pl.loop(D, NCHUNK - D, step=NBUF)
    def _(j0):
        for b in range(NBUF):
            j = j0 + b
            buf = (D + b) % NBUF
            wait_gather(j, buf)
            scatter(j, buf)
            bg = (D + b + D) % NBUF
            wait_scatter(j - D, bg)
            gather(j + D, bg)

    for j in range(NCHUNK - D, NCHUNK):
        wait_gather(j, j % NBUF)
        scatter(j, j % NBUF)
    for j in range(NCHUNK - NBUF, NCHUNK):
        wait_scatter(j, j % NBUF)

    plsc.subcore_barrier()
    pltpu.sync_copy(acc_sh.at[pl.ds(sid * RPT, RPT)],
                    out_hbm.at[pl.ds(cid * N_P + sid * RPT, RPT)])


@functools.partial(
    pl.kernel,
    out_type=jax.ShapeDtypeStruct((NC * N_P, D_HID), _f32),
    mesh=_mesh,
    scratch_types=[
        pltpu.VMEM((NCHUNK, C), _i32),
        pltpu.VMEM((C, D_HID), _f32),
        [pltpu.SemaphoreType.DMA for _ in range(4)],
        pltpu.VMEM_SHARED((N_P, D_HID), _f32),
    ],
    compiler_params=_sc_params,
)
def _count(dst_hbm, out_hbm, dst_v, rows_v, csem, acc_sh):
    cid = lax.axis_index("c")
    sid = lax.axis_index("s")
    tid = cid * NS + sid

    @pl.loop(0, C)
    def _(i):
        rows_v[i, :] = jnp.zeros((D_HID,), _f32)

    @pl.loop(0, RPT // C)
    def _(k):
        pltpu.sync_copy(rows_v, acc_sh.at[pl.ds(sid * RPT + k * C, C)])

    @pl.loop(0, C)
    def _(i):
        rows_v[i, :] = jnp.ones((D_HID,), _f32)

    pltpu.sync_copy(dst_hbm.at[pl.ds(tid * NCHUNK, NCHUNK)], dst_v)
    plsc.subcore_barrier()

    def scatter(j, b):
        pltpu.async_copy(rows_v, acc_sh.at[dst_v.at[j]], csem[b], add=True)

    def wait_scatter(j, b):
        pltpu.make_async_copy(rows_v, acc_sh.at[dst_v.at[j]], csem[b]).wait()

    for j in range(4):
        scatter(j, j)

    @pl.loop(4, NCHUNK, step=4)
    def _(j0):
        for b in range(4):
            j = j0 + b
            wait_scatter(j - 4, b)
            scatter(j, b)

    for j in range(NCHUNK - 4, NCHUNK):
        wait_scatter(j, j % 4)

    plsc.subcore_barrier()
    pltpu.sync_copy(acc_sh.at[pl.ds(sid * RPT, RPT)],
                    out_hbm.at[pl.ds(cid * N_P + sid * RPT, RPT)])


EPB = 81920
NBLK = E_PAD // EPB


def _prep_body(e_ref, src_ref, dst_ref):
    i = pl.program_id(0)
    s = e_ref[0].reshape(EPB // C, C)
    d = e_ref[1].reshape(EPB // C, C)
    row = lax.broadcasted_iota(_i32, (EPB // C, C), 0)
    lane = lax.broadcasted_iota(_i32, (EPB // C, C), 1)
    eid = i * EPB + row * C + lane
    valid = eid < E
    src_ref[...] = jnp.where(valid, s, eid & 8191)
    dst_ref[...] = jnp.where(valid, d, N + (eid & 127))


def _prep(edge_index):
    return pl.pallas_call(
        _prep_body,
        grid=(NBLK,),
        in_specs=[pl.BlockSpec((2, EPB), lambda i: (0, i))],
        out_specs=[pl.BlockSpec((EPB // C, C), lambda i: (i, 0)),
                   pl.BlockSpec((EPB // C, C), lambda i: (i, 0))],
        out_shape=[jax.ShapeDtypeStruct((NW * NCHUNK, C), _i32),
                   jax.ShapeDtypeStruct((NW * NCHUNK, C), _i32)],
    )(edge_index)


def _full(shape):
    return pl.BlockSpec(shape, lambda i: tuple(0 for _ in shape))


def _p2():
    return [pl.BlockSpec((NF, W128), lambda i: (0, 0)),
            pl.BlockSpec((NF, W128), lambda i: (1, 0))]


def _mm1_body(xg_ref, wbd_ref, o_ref):
    o_ref[...] = jnp.dot(xg_ref[...], wbd_ref[...], preferred_element_type=_f32)


def _matmul1(x_g, W_bd):
    return pl.pallas_call(
        _mm1_body,
        grid=(1,),
        in_specs=[_full((NF, G * D_IN)), _full((G * D_IN, W128))],
        out_specs=_full((NF, W128)),
        out_shape=jax.ShapeDtypeStruct((NF, W128), _f32),
    )(x_g, W_bd)


def _scale1_body(cnt0_ref, cnt1_ref, h1_ref, hs_ref, dinv_ref):
    dinv = lax.rsqrt(cnt0_ref[...] + cnt1_ref[...] + 1.0)
    dinv_ref[...] = dinv
    hs_ref[...] = dinv * h1_ref[...]


def _scale1(cnt_f, h1_f):
    return pl.pallas_call(
        _scale1_body,
        grid=(1,),
        in_specs=_p2() + [_full((NF, W128))],
        out_specs=[_full((NF, W128)), _full((NF, W128))],
        out_shape=[jax.ShapeDtypeStruct((NF, W128), _f32),
                   jax.ShapeDtypeStruct((NF, W128), _f32)],
    )(cnt_f, cnt_f, h1_f)


def _scale2_body(s10_ref, s11_ref, h1s_ref, dinv_ref, b1_ref, o_ref):
    out1 = (dinv_ref[...] * (s10_ref[...] + s11_ref[...] + h1s_ref[...])
            + b1_ref[...])
    o_ref[...] = dinv_ref[...] * jnp.maximum(out1, 0.0)


def _scale2(s1_f, h1s_f, dinv_f, b1_t):
    return pl.pallas_call(
        _scale2_body,
        grid=(1,),
        in_specs=_p2() + [_full((NF, W128)), _full((NF, W128)),
                          _full((1, W128))],
        out_specs=_full((NF, W128)),
        out_shape=jax.ShapeDtypeStruct((NF, W128), _f32),
    )(s1_f, s1_f, h1s_f, dinv_f, b1_t)


def _final_body(s20_ref, s21_ref, hs_ref, dinv_ref, w2bd_ref, b2_ref, o_ref):
    u = dinv_ref[...] * (s20_ref[...] + s21_ref[...] + hs_ref[...])
    o_ref[...] = (jnp.dot(u, w2bd_ref[...], preferred_element_type=_f32)
                  + b2_ref[...])


def _final(s2_f, hs_f, dinv_f, W2_bd, b2_t):
    return pl.pallas_call(
        _final_body,
        grid=(1,),
        in_specs=_p2() + [_full((NF, W128)), _full((NF, W128)),
                          _full((W128, G * D_OUT)), _full((1, G * D_OUT))],
        out_specs=_full((NF, G * D_OUT)),
        out_shape=jax.ShapeDtypeStruct((NF, G * D_OUT), _f32),
    )(s2_f, s2_f, hs_f, dinv_f, W2_bd, b2_t)


def _block_diag(W, g):
    a, b = W.shape
    eye = jnp.eye(g, dtype=W.dtype)
    return (eye[:, None, :, None] * W[None, :, None, :]).reshape(g * a, g * b)


def kernel(x, edge_index, W1, b1, W2, b2):
    x = x.astype(_f32)
    src_p, dst_p = _prep(edge_index.astype(_i32))
    x_g = jnp.pad(x.reshape(N // G, G * D_IN), ((0, NF - N // G), (0, 0)))
    W_bd = _block_diag(W1, G)
    W2_bd = _block_diag(W2, G)
    b1_t = jnp.tile(b1, G).reshape(1, W128)
    b2_t = jnp.tile(b2, G).reshape(1, G * D_OUT)

    cnt = _count(dst_p)
    cnt_f = cnt.reshape(2 * NF, W128)
    h1_f = _matmul1(x_g, W_bd)
    h1s_f, dinv_f = _scale1(cnt_f, h1_f)
    s1 = _seg_sum(h1s_f.reshape(N_P, D_HID), src_p, dst_p)
    hs_f = _scale2(s1.reshape(2 * NF, W128), h1s_f, dinv_f, b1_t)
    s2 = _seg_sum(hs_f.reshape(N_P, D_HID), src_p, dst_p)
    out_g = _final(s2.reshape(2 * NF, W128), hs_f, dinv_f, W2_bd, b2_t)
    return out_g.reshape(N_P, D_OUT)[:N]


# speedup vs baseline: 1.9146x; 1.0243x over previous
import functools

import jax
import jax.numpy as jnp
from jax import lax
from jax.experimental import pallas as pl
from jax.experimental.pallas import tpu as pltpu
from jax.experimental.pallas import tpu_sc as plsc

N = 10000
E = 320000
D_IN = 128
D_HID = 16
D_OUT = 128

NC = 2
NS = 16
NW = NC * NS
C = 128
NCHUNK = 80
EPT = NCHUNK * C
E_PAD = EPT * NW
N_P = 10240
RPT = N_P // NS
G = 8
NF = N_P // G
W128 = G * D_HID

_f32 = jnp.float32
_i32 = jnp.int32

_mesh = plsc.VectorSubcoreMesh(core_axis_name="c", subcore_axis_name="s",
                               num_cores=NC, num_subcores=NS)
_sc_params = pltpu.CompilerParams(use_tc_tiling_on_sc=False)


NBUF = 8


@functools.partial(
    pl.kernel,
    out_type=jax.ShapeDtypeStruct((NC * N_P, D_HID), _f32),
    mesh=_mesh,
    scratch_types=[
        pltpu.VMEM((NCHUNK, C), _i32),
        pltpu.VMEM((NCHUNK, C), _i32),
        [pltpu.VMEM((C, D_HID), _f32) for _ in range(NBUF)],
        [pltpu.SemaphoreType.DMA for _ in range(NBUF)],
        [pltpu.SemaphoreType.DMA for _ in range(NBUF)],
        pltpu.VMEM_SHARED((N_P, D_HID), _f32),
    ],
    compiler_params=_sc_params,
)
def _seg_sum(table_hbm, src_hbm, dst_hbm, out_hbm,
             src_v, dst_v, rows, gsem, ssem, acc_sh):
    cid = lax.axis_index("c")
    sid = lax.axis_index("s")
    tid = cid * NS + sid

    @pl.loop(0, C)
    def _(i):
        rows[0][i, :] = jnp.zeros((D_HID,), _f32)

    @pl.loop(0, RPT // C)
    def _(k):
        pltpu.sync_copy(rows[0], acc_sh.at[pl.ds(sid * RPT + k * C, C)])

    pltpu.sync_copy(src_hbm.at[pl.ds(tid * NCHUNK, NCHUNK)], src_v)
    pltpu.sync_copy(dst_hbm.at[pl.ds(tid * NCHUNK, NCHUNK)], dst_v)
    plsc.subcore_barrier()

    def gather(j, b):
        pltpu.async_copy(table_hbm.at[src_v.at[j]], rows[b], gsem[b])

    def wait_gather(j, b):
        pltpu.make_async_copy(table_hbm.at[src_v.at[j]], rows[b],
                              gsem[b]).wait()

    def scatter(j, b):
        pltpu.async_copy(rows[b], acc_sh.at[dst_v.at[j]], ssem[b], add=True)

    def wait_scatter(j, b):
        pltpu.make_async_copy(rows[b], acc_sh.at[dst_v.at[j]],
                              ssem[b]).wait()

    D = NBUF // 2
    for j in range(D):
        gather(j, j)
    for j in range(D):
        wait_gather(j, j)
        scatter(j, j)
        gather(j + D, j + D)

    @pl.loop(D, NCHUNK - D, step=NBUF)
    def _(j0):
        for b in range(NBUF):
            j = j0 + b
            buf = (D + b) % NBUF
            wait_gather(j, buf)
            scatter(j, buf)
            bg = (D + b + D) % NBUF
            wait_scatter(j - D, bg)
            gather(j + D, bg)

    for j in range(NCHUNK - D, NCHUNK):
        wait_gather(j, j % NBUF)
        scatter(j, j % NBUF)
    for j in range(NCHUNK - NBUF, NCHUNK):
        wait_scatter(j, j % NBUF)

    plsc.subcore_barrier()
    pltpu.sync_copy(acc_sh.at[pl.ds(sid * RPT, RPT)],
                    out_hbm.at[pl.ds(cid * N_P + sid * RPT, RPT)])


@functools.partial(
    pl.kernel,
    out_type=jax.ShapeDtypeStruct((NC * N_P, D_HID), _f32),
    mesh=_mesh,
    scratch_types=[
        pltpu.VMEM((NCHUNK, C), _i32),
        pltpu.VMEM((C, D_HID), _f32),
        [pltpu.SemaphoreType.DMA for _ in range(8)],
        pltpu.VMEM_SHARED((N_P, D_HID), _f32),
    ],
    compiler_params=_sc_params,
)
def _count(dst_hbm, out_hbm, dst_v, rows_v, csem, acc_sh):
    cid = lax.axis_index("c")
    sid = lax.axis_index("s")
    tid = cid * NS + sid

    @pl.loop(0, C)
    def _(i):
        rows_v[i, :] = jnp.zeros((D_HID,), _f32)

    @pl.loop(0, RPT // C)
    def _(k):
        pltpu.sync_copy(rows_v, acc_sh.at[pl.ds(sid * RPT + k * C, C)])

    @pl.loop(0, C)
    def _(i):
        rows_v[i, :] = jnp.ones((D_HID,), _f32)

    pltpu.sync_copy(dst_hbm.at[pl.ds(tid * NCHUNK, NCHUNK)], dst_v)
    plsc.subcore_barrier()

    def scatter(j, b):
        pltpu.async_copy(rows_v, acc_sh.at[dst_v.at[j]], csem[b], add=True)

    def wait_scatter(j, b):
        pltpu.make_async_copy(rows_v, acc_sh.at[dst_v.at[j]], csem[b]).wait()

    for j in range(8):
        scatter(j, j)

    @pl.loop(8, NCHUNK, step=8)
    def _(j0):
        for b in range(8):
            j = j0 + b
            wait_scatter(j - 8, b)
            scatter(j, b)

    for j in range(NCHUNK - 8, NCHUNK):
        wait_scatter(j, j % 8)

    plsc.subcore_barrier()
    pltpu.sync_copy(acc_sh.at[pl.ds(sid * RPT, RPT)],
                    out_hbm.at[pl.ds(cid * N_P + sid * RPT, RPT)])


EPB = 81920
NBLK = E_PAD // EPB


def _prep_body(e_ref, src_ref, dst_ref):
    i = pl.program_id(0)
    s = e_ref[0].reshape(EPB // C, C)
    d = e_ref[1].reshape(EPB // C, C)
    row = lax.broadcasted_iota(_i32, (EPB // C, C), 0)
    lane = lax.broadcasted_iota(_i32, (EPB // C, C), 1)
    eid = i * EPB + row * C + lane
    valid = eid < E
    src_ref[...] = jnp.where(valid, s, eid & 8191)
    dst_ref[...] = jnp.where(valid, d, N + (eid & 127))


def _prep(edge_index):
    return pl.pallas_call(
        _prep_body,
        grid=(NBLK,),
        in_specs=[pl.BlockSpec((2, EPB), lambda i: (0, i))],
        out_specs=[pl.BlockSpec((EPB // C, C), lambda i: (i, 0)),
                   pl.BlockSpec((EPB // C, C), lambda i: (i, 0))],
        out_shape=[jax.ShapeDtypeStruct((NW * NCHUNK, C), _i32),
                   jax.ShapeDtypeStruct((NW * NCHUNK, C), _i32)],
    )(edge_index)


def _full(shape):
    return pl.BlockSpec(shape, lambda i: tuple(0 for _ in shape))


def _p2():
    return [pl.BlockSpec((NF, W128), lambda i: (0, 0)),
            pl.BlockSpec((NF, W128), lambda i: (1, 0))]


def _mm1_body(xg_ref, wbd_ref, o_ref):
    o_ref[...] = jnp.dot(xg_ref[...], wbd_ref[...], preferred_element_type=_f32)


def _matmul1(x_g, W_bd):
    return pl.pallas_call(
        _mm1_body,
        grid=(1,),
        in_specs=[_full((NF, G * D_IN)), _full((G * D_IN, W128))],
        out_specs=_full((NF, W128)),
        out_shape=jax.ShapeDtypeStruct((NF, W128), _f32),
    )(x_g, W_bd)


def _scale1_body(cnt0_ref, cnt1_ref, h1_ref, hs_ref, dinv_ref):
    dinv = lax.rsqrt(cnt0_ref[...] + cnt1_ref[...] + 1.0)
    dinv_ref[...] = dinv
    hs_ref[...] = dinv * h1_ref[...]


def _scale1(cnt_f, h1_f):
    return pl.pallas_call(
        _scale1_body,
        grid=(1,),
        in_specs=_p2() + [_full((NF, W128))],
        out_specs=[_full((NF, W128)), _full((NF, W128))],
        out_shape=[jax.ShapeDtypeStruct((NF, W128), _f32),
                   jax.ShapeDtypeStruct((NF, W128), _f32)],
    )(cnt_f, cnt_f, h1_f)


def _scale2_body(s10_ref, s11_ref, h1s_ref, dinv_ref, b1_ref, o_ref):
    out1 = (dinv_ref[...] * (s10_ref[...] + s11_ref[...] + h1s_ref[...])
            + b1_ref[...])
    o_ref[...] = dinv_ref[...] * jnp.maximum(out1, 0.0)


def _scale2(s1_f, h1s_f, dinv_f, b1_t):
    return pl.pallas_call(
        _scale2_body,
        grid=(1,),
        in_specs=_p2() + [_full((NF, W128)), _full((NF, W128)),
                          _full((1, W128))],
        out_specs=_full((NF, W128)),
        out_shape=jax.ShapeDtypeStruct((NF, W128), _f32),
    )(s1_f, s1_f, h1s_f, dinv_f, b1_t)


FBLK = 128


def _final_body(s20_ref, s21_ref, hs_ref, dinv_ref, w2_ref, b2_ref, o_ref):
    u_f = dinv_ref[...] * (s20_ref[...] + s21_ref[...] + hs_ref[...])
    for j in range(G):
        uj = u_f[:, j * D_HID:(j + 1) * D_HID]
        o_ref[j::G, :] = (jnp.dot(uj, w2_ref[...], preferred_element_type=_f32)
                          + b2_ref[...])


def _final(s2_f, hs_f, dinv_f, W2, b2_r):
    return pl.pallas_call(
        _final_body,
        grid=(NF // FBLK,),
        in_specs=[pl.BlockSpec((FBLK, W128), lambda i: (i, 0)),
                  pl.BlockSpec((FBLK, W128), lambda i: (NF // FBLK + i, 0)),
                  pl.BlockSpec((FBLK, W128), lambda i: (i, 0)),
                  pl.BlockSpec((FBLK, W128), lambda i: (i, 0)),
                  pl.BlockSpec((D_HID, D_OUT), lambda i: (0, 0)),
                  pl.BlockSpec((1, D_OUT), lambda i: (0, 0))],
        out_specs=pl.BlockSpec((FBLK * G, D_OUT), lambda i: (i, 0)),
        out_shape=jax.ShapeDtypeStruct((N, D_OUT), _f32),
    )(s2_f, s2_f, hs_f, dinv_f, W2, b2_r)


def _block_diag(W, g):
    a, b = W.shape
    eye = jnp.eye(g, dtype=W.dtype)
    return (eye[:, None, :, None] * W[None, :, None, :]).reshape(g * a, g * b)


def kernel(x, edge_index, W1, b1, W2, b2):
    x = x.astype(_f32)
    src_p, dst_p = _prep(edge_index.astype(_i32))
    x_g = jnp.pad(x.reshape(N // G, G * D_IN), ((0, NF - N // G), (0, 0)))
    W_bd = _block_diag(W1, G)
    b1_t = jnp.tile(b1, G).reshape(1, W128)

    cnt = _count(dst_p)
    cnt_f = cnt.reshape(2 * NF, W128)
    h1_f = _matmul1(x_g, W_bd)
    h1s_f, dinv_f = _scale1(cnt_f, h1_f)
    s1 = _seg_sum(h1s_f.reshape(N_P, D_HID), src_p, dst_p)
    hs_f = _scale2(s1.reshape(2 * NF, W128), h1s_f, dinv_f, b1_t)
    s2 = _seg_sum(hs_f.reshape(N_P, D_HID), src_p, dst_p)
    return _final(s2.reshape(2 * NF, W128), hs_f, dinv_f, W2,
                  b2.reshape(1, D_OUT))


# speedup vs baseline: 2.0141x; 1.0520x over previous
import functools

import jax
import jax.numpy as jnp
from jax import lax
from jax.experimental import pallas as pl
from jax.experimental.pallas import tpu as pltpu
from jax.experimental.pallas import tpu_sc as plsc

N = 10000
E = 320000
D_IN = 128
D_HID = 16
D_OUT = 128

NC = 2
NS = 16
NW = NC * NS
C = 128
NCHUNK = 80
EPT = NCHUNK * C
E_PAD = EPT * NW
N_P = 10240
RPT = N_P // NS
G = 8
NF = N_P // G
W128 = G * D_HID

_f32 = jnp.float32
_i32 = jnp.int32

_mesh = plsc.VectorSubcoreMesh(core_axis_name="c", subcore_axis_name="s",
                               num_cores=NC, num_subcores=NS)
_sc_params = pltpu.CompilerParams(use_tc_tiling_on_sc=False)


NBUF = 10


@functools.partial(
    pl.kernel,
    out_type=jax.ShapeDtypeStruct((NC * N_P, D_HID), _f32),
    mesh=_mesh,
    scratch_types=[
        pltpu.VMEM((NCHUNK, C), _i32),
        pltpu.VMEM((NCHUNK, C), _i32),
        [pltpu.VMEM((C, D_HID), _f32) for _ in range(NBUF)],
        [pltpu.SemaphoreType.DMA for _ in range(NBUF)],
        [pltpu.SemaphoreType.DMA for _ in range(NBUF)],
        pltpu.VMEM_SHARED((N_P, D_HID), _f32),
    ],
    compiler_params=_sc_params,
)
def _seg_sum(table_hbm, src_hbm, dst_hbm, out_hbm,
             src_v, dst_v, rows, gsem, ssem, acc_sh):
    cid = lax.axis_index("c")
    sid = lax.axis_index("s")
    tid = cid * NS + sid

    @pl.loop(0, C)
    def _(i):
        rows[0][i, :] = jnp.zeros((D_HID,), _f32)

    @pl.loop(0, RPT // C)
    def _(k):
        pltpu.sync_copy(rows[0], acc_sh.at[pl.ds(sid * RPT + k * C, C)])

    pltpu.sync_copy(src_hbm.at[pl.ds(tid * NCHUNK, NCHUNK)], src_v)
    pltpu.sync_copy(dst_hbm.at[pl.ds(tid * NCHUNK, NCHUNK)], dst_v)
    plsc.subcore_barrier()

    def gather(j, b):
        pltpu.async_copy(table_hbm.at[src_v.at[j]], rows[b], gsem[b])

    def wait_gather(j, b):
        pltpu.make_async_copy(table_hbm.at[src_v.at[j]], rows[b],
                              gsem[b]).wait()

    def scatter(j, b):
        pltpu.async_copy(rows[b], acc_sh.at[dst_v.at[j]], ssem[b], add=True)

    def wait_scatter(j, b):
        pltpu.make_async_copy(rows[b], acc_sh.at[dst_v.at[j]],
                              ssem[b]).wait()

    D = NBUF // 2
    for j in range(D):
        gather(j, j)
    for j in range(D):
        wait_gather(j, j)
        scatter(j, j)
        gather(j + D, j + D)

    @pl.loop(D, NCHUNK - D, step=NBUF)
    def _(j0):
        for b in range(NBUF):
            j = j0 + b
            buf = (D + b) % NBUF
            wait_gather(j, buf)
            scatter(j, buf)
            bg = (D + b + D) % NBUF
            wait_scatter(j - D, bg)
            gather(j + D, bg)

    for j in range(NCHUNK - D, NCHUNK):
        wait_gather(j, j % NBUF)
        scatter(j, j % NBUF)
    for j in range(NCHUNK - NBUF, NCHUNK):
        wait_scatter(j, j % NBUF)

    plsc.subcore_barrier()
    pltpu.sync_copy(acc_sh.at[pl.ds(sid * RPT, RPT)],
                    out_hbm.at[pl.ds(cid * N_P + sid * RPT, RPT)])


@functools.partial(
    pl.kernel,
    out_type=jax.ShapeDtypeStruct((NC * N_P, D_HID), _f32),
    mesh=_mesh,
    scratch_types=[
        pltpu.VMEM((NCHUNK, C), _i32),
        pltpu.VMEM((C, D_HID), _f32),
        [pltpu.SemaphoreType.DMA for _ in range(8)],
        pltpu.VMEM_SHARED((N_P, D_HID), _f32),
    ],
    compiler_params=_sc_params,
)
def _count(dst_hbm, out_hbm, dst_v, rows_v, csem, acc_sh):
    cid = lax.axis_index("c")
    sid = lax.axis_index("s")
    tid = cid * NS + sid

    @pl.loop(0, C)
    def _(i):
        rows_v[i, :] = jnp.zeros((D_HID,), _f32)

    @pl.loop(0, RPT // C)
    def _(k):
        pltpu.sync_copy(rows_v, acc_sh.at[pl.ds(sid * RPT + k * C, C)])

    @pl.loop(0, C)
    def _(i):
        rows_v[i, :] = jnp.ones((D_HID,), _f32)

    pltpu.sync_copy(dst_hbm.at[pl.ds(tid * NCHUNK, NCHUNK)], dst_v)
    plsc.subcore_barrier()

    def scatter(j, b):
        pltpu.async_copy(rows_v, acc_sh.at[dst_v.at[j]], csem[b], add=True)

    def wait_scatter(j, b):
        pltpu.make_async_copy(rows_v, acc_sh.at[dst_v.at[j]], csem[b]).wait()

    for j in range(8):
        scatter(j, j)

    @pl.loop(8, NCHUNK, step=8)
    def _(j0):
        for b in range(8):
            j = j0 + b
            wait_scatter(j - 8, b)
            scatter(j, b)

    for j in range(NCHUNK - 8, NCHUNK):
        wait_scatter(j, j % 8)

    plsc.subcore_barrier()
    pltpu.sync_copy(acc_sh.at[pl.ds(sid * RPT, RPT)],
                    out_hbm.at[pl.ds(cid * N_P + sid * RPT, RPT)])


EPB = 81920
NBLK = E_PAD // EPB


def _prep_body(e_ref, src_ref, dst_ref):
    i = pl.program_id(0)
    s = e_ref[0].reshape(EPB // C, C)
    d = e_ref[1].reshape(EPB // C, C)
    row = lax.broadcasted_iota(_i32, (EPB // C, C), 0)
    lane = lax.broadcasted_iota(_i32, (EPB // C, C), 1)
    eid = i * EPB + row * C + lane
    valid = eid < E
    src_ref[...] = jnp.where(valid, s, eid & 8191)
    dst_ref[...] = jnp.where(valid, d, N + (eid & 127))


def _prep(edge_index):
    return pl.pallas_call(
        _prep_body,
        grid=(NBLK,),
        in_specs=[pl.BlockSpec((2, EPB), lambda i: (0, i))],
        out_specs=[pl.BlockSpec((EPB // C, C), lambda i: (i, 0)),
                   pl.BlockSpec((EPB // C, C), lambda i: (i, 0))],
        out_shape=[jax.ShapeDtypeStruct((NW * NCHUNK, C), _i32),
                   jax.ShapeDtypeStruct((NW * NCHUNK, C), _i32)],
    )(edge_index)


def _full(shape):
    return pl.BlockSpec(shape, lambda i: tuple(0 for _ in shape))


def _p2():
    return [pl.BlockSpec((NF, W128), lambda i: (0, 0)),
            pl.BlockSpec((NF, W128), lambda i: (1, 0))]


def _mm1_body(xg_ref, wbd_ref, o_ref):
    o_ref[...] = jnp.dot(xg_ref[...], wbd_ref[...], preferred_element_type=_f32)


def _matmul1(x_g, W_bd):
    return pl.pallas_call(
        _mm1_body,
        grid=(1,),
        in_specs=[_full((NF, G * D_IN)), _full((G * D_IN, W128))],
        out_specs=_full((NF, W128)),
        out_shape=jax.ShapeDtypeStruct((NF, W128), _f32),
    )(x_g, W_bd)


def _scale1_body(cnt0_ref, cnt1_ref, h1_ref, hs_ref, dinv_ref):
    dinv = lax.rsqrt(cnt0_ref[...] + cnt1_ref[...] + 1.0)
    dinv_ref[...] = dinv
    hs_ref[...] = dinv * h1_ref[...]


def _scale1(cnt_f, h1_f):
    return pl.pallas_call(
        _scale1_body,
        grid=(1,),
        in_specs=_p2() + [_full((NF, W128))],
        out_specs=[_full((NF, W128)), _full((NF, W128))],
        out_shape=[jax.ShapeDtypeStruct((NF, W128), _f32),
                   jax.ShapeDtypeStruct((NF, W128), _f32)],
    )(cnt_f, cnt_f, h1_f)


def _scale2_body(s10_ref, s11_ref, h1s_ref, dinv_ref, b1_ref, o_ref):
    out1 = (dinv_ref[...] * (s10_ref[...] + s11_ref[...] + h1s_ref[...])
            + b1_ref[...])
    o_ref[...] = dinv_ref[...] * jnp.maximum(out1, 0.0)


def _scale2(s1_f, h1s_f, dinv_f, b1_t):
    return pl.pallas_call(
        _scale2_body,
        grid=(1,),
        in_specs=_p2() + [_full((NF, W128)), _full((NF, W128)),
                          _full((1, W128))],
        out_specs=_full((NF, W128)),
        out_shape=jax.ShapeDtypeStruct((NF, W128), _f32),
    )(s1_f, s1_f, h1s_f, dinv_f, b1_t)


FBLK = 128


def _final_body(s20_ref, s21_ref, hs_ref, dinv_ref, w2_ref, b2_ref, o_ref):
    u_f = dinv_ref[...] * (s20_ref[...] + s21_ref[...] + hs_ref[...])
    for j in range(G):
        uj = u_f[:, j * D_HID:(j + 1) * D_HID]
        o_ref[j::G, :] = (jnp.dot(uj, w2_ref[...], preferred_element_type=_f32)
                          + b2_ref[...])


def _final(s2_f, hs_f, dinv_f, W2, b2_r):
    return pl.pallas_call(
        _final_body,
        grid=(NF // FBLK,),
        in_specs=[pl.BlockSpec((FBLK, W128), lambda i: (i, 0)),
                  pl.BlockSpec((FBLK, W128), lambda i: (NF // FBLK + i, 0)),
                  pl.BlockSpec((FBLK, W128), lambda i: (i, 0)),
                  pl.BlockSpec((FBLK, W128), lambda i: (i, 0)),
                  pl.BlockSpec((D_HID, D_OUT), lambda i: (0, 0)),
                  pl.BlockSpec((1, D_OUT), lambda i: (0, 0))],
        out_specs=pl.BlockSpec((FBLK * G, D_OUT), lambda i: (i, 0)),
        out_shape=jax.ShapeDtypeStruct((N, D_OUT), _f32),
    )(s2_f, s2_f, hs_f, dinv_f, W2, b2_r)


def _block_diag(W, g):
    a, b = W.shape
    eye = jnp.eye(g, dtype=W.dtype)
    return (eye[:, None, :, None] * W[None, :, None, :]).reshape(g * a, g * b)


def kernel(x, edge_index, W1, b1, W2, b2):
    x = x.astype(_f32)
    src_p, dst_p = _prep(edge_index.astype(_i32))
    x_g = jnp.pad(x.reshape(N // G, G * D_IN), ((0, NF - N // G), (0, 0)))
    W_bd = _block_diag(W1, G)
    b1_t = jnp.tile(b1, G).reshape(1, W128)

    cnt = _count(dst_p)
    cnt_f = cnt.reshape(2 * NF, W128)
    h1_f = _matmul1(x_g, W_bd)
    h1s_f, dinv_f = _scale1(cnt_f, h1_f)
    s1 = _seg_sum(h1s_f.reshape(N_P, D_HID), src_p, dst_p)
    hs_f = _scale2(s1.reshape(2 * NF, W128), h1s_f, dinv_f, b1_t)
    s2 = _seg_sum(hs_f.reshape(N_P, D_HID), src_p, dst_p)
    return _final(s2.reshape(2 * NF, W128), hs_f, dinv_f, W2,
                  b2.reshape(1, D_OUT))
